# Initial kernel scaffold; baseline (speedup 1.0000x reference)
import jax
import jax.numpy as jnp
from jax.experimental import pallas as pl


def kernel(query, key, value, Wqi, Wki, head_weights, Wqkv, bqkv, Wout, bout):
    raise NotImplementedError("write your pallas kernel here")


# speedup vs baseline: 3.5906x; 3.5906x over previous
import functools
import math

import jax
import jax.numpy as jnp
from jax import lax
from jax.experimental import pallas as pl
from jax.experimental.pallas import tpu as pltpu
from jax.experimental.pallas import tpu_sc as plsc

_B, _TQ, _TK, _C = 4, 16, 8192, 1024
_H, _DH = 16, 64
_HI, _DI = 32, 64
_TOPK = 1024

_BM = 512


def _proj_q_body(q_ref, wqiT_ref, wQT_ref, bq_ref, iq_ref, qo_ref):
    q = q_ref[...]
    iq_ref[...] = jnp.dot(q, wqiT_ref[...], preferred_element_type=jnp.float32)
    qo_ref[...] = (
        jnp.dot(q, wQT_ref[...], preferred_element_type=jnp.float32) + bq_ref[...]
    )


def _proj_q(qf, wqiT, wQT, bq):
    return pl.pallas_call(
        _proj_q_body,
        out_shape=(
            jax.ShapeDtypeStruct((_B * _TQ, _HI * _DI), jnp.float32),
            jax.ShapeDtypeStruct((_B * _TQ, _C), jnp.float32),
        ),
    )(qf, wqiT, wQT, bq)


def _proj_k_body(x_ref, wkvT_ref, bkv_ref, wkiT_ref, iq_ref, w_ref, kv_ref, s_ref):
    x = x_ref[0]
    kv_ref[0] = (
        jnp.dot(x, wkvT_ref[...], preferred_element_type=jnp.float32) + bkv_ref[...]
    )
    ik = jnp.dot(x, wkiT_ref[...], preferred_element_type=jnp.float32)
    acc = jnp.zeros((_TQ, _BM), jnp.float32)
    for h in range(_HI):
        iq_h = iq_ref[0, h]
        ik_h = ik[:, h * _DI : (h + 1) * _DI]
        s = lax.dot_general(
            iq_h, ik_h, (((1,), (1,)), ((), ())), preferred_element_type=jnp.float32
        )
        v = jnp.maximum(s * (1.0 / math.sqrt(_DI)), 0.0)
        vq = v.astype(jnp.bfloat16).astype(jnp.float32)
        acc = acc + vq * w_ref[h]
    s_ref[0] = acc


def _proj_k(key, wkvT, bkv, wkiT, iq_r, w_q):
    grid = (_B, _TK // _BM)
    return pl.pallas_call(
        _proj_k_body,
        grid=grid,
        in_specs=[
            pl.BlockSpec((1, _BM, _C), lambda b, m: (b, m, 0)),
            pl.BlockSpec((_C, 2 * _C), lambda b, m: (0, 0)),
            pl.BlockSpec((1, 2 * _C), lambda b, m: (0, 0)),
            pl.BlockSpec((_C, _HI * _DI), lambda b, m: (0, 0)),
            pl.BlockSpec((1, _HI, _TQ, _DI), lambda b, m: (b, 0, 0, 0)),
            pl.BlockSpec(memory_space=pltpu.SMEM),
        ],
        out_specs=(
            pl.BlockSpec((1, _BM, 2 * _C), lambda b, m: (b, m, 0)),
            pl.BlockSpec((1, _TQ, _BM), lambda b, m: (b, 0, m)),
        ),
        out_shape=(
            jax.ShapeDtypeStruct((_B, _TK, 2 * _C), jnp.float32),
            jax.ShapeDtypeStruct((_B, _TQ, _TK), jnp.float32),
        ),
        compiler_params=pltpu.CompilerParams(
            dimension_semantics=("parallel", "parallel"),
        ),
    )(key, wkvT, bkv, wkiT, iq_r, w_q)


def _topk_sc_body(sbits_ref, out_ref, rows_v, hist_v, out_v):
    c = lax.axis_index("c")
    s = lax.axis_index("s")
    wid = c * 16 + s
    base = wid * 2
    pltpu.sync_copy(sbits_ref.at[pl.ds(base, 2)], rows_v)
    lanes = lax.iota(jnp.int32, 16)
    ones = jnp.ones((16,), jnp.int32)
    zeros16 = jnp.zeros((16,), jnp.int32)
    res = jnp.zeros((16,), jnp.int32)
    for r in range(2):
        prefix = jnp.int32(0)
        krem = jnp.int32(_TOPK)
        for sh, nb in ((23, 8), (15, 8), (7, 8), (0, 7)):
            mask_d = (1 << nb) - 1
            hb = sh + nb
            for cc in range(16):
                hist_v[pl.ds(cc * 16, 16)] = zeros16

            def body(i, carry, r=r, sh=sh, hb=hb, mask_d=mask_d, prefix=prefix):
                x = rows_v[r, pl.ds(i * 16, 16)]
                digit = lax.shift_right_logical(x, sh) & mask_d
                match = lax.shift_right_logical(x, hb) == prefix
                ridx = mask_d - digit
                plsc.addupdate_scatter(hist_v, [ridx], ones, mask=match)
                return carry

            lax.fori_loop(0, _TK // 16, body, 0, unroll=8)

            tot = jnp.int32(0)
            found = jnp.bool_(False)
            digit_sel = jnp.int32(0)
            above_sel = jnp.int32(0)
            for cc in range(16):
                hv = hist_v[pl.ds(cc * 16, 16)]
                cs = plsc.cumsum(hv) + tot
                ge = cs >= krem
                has = jnp.any(ge)
                pc = plsc.cumsum(ge.astype(jnp.int32))
                fm = ge & (pc == 1)
                cum_at = jnp.sum(jnp.where(fm, cs, 0))
                h_at = jnp.sum(jnp.where(fm, hv, 0))
                ridx_at = jnp.sum(jnp.where(fm, lanes, 0)) + cc * 16
                take = has & (~found)
                digit_sel = jnp.where(take, mask_d - ridx_at, digit_sel)
                above_sel = jnp.where(take, cum_at - h_at, above_sel)
                found = found | has
                tot = tot + jnp.sum(hv)
            krem = krem - above_sel
            prefix = jnp.bitwise_or(lax.shift_left(prefix, nb), digit_sel)
        res = jnp.where(lanes == r, prefix, res)
    out_v[...] = res
    pltpu.sync_copy(out_v, out_ref.at[wid])


def _topk_sc(sbits):
    mesh = plsc.VectorSubcoreMesh(core_axis_name="c", subcore_axis_name="s")
    return pl.kernel(
        _topk_sc_body,
        out_type=jax.ShapeDtypeStruct((32, 16), jnp.int32),
        mesh=mesh,
        scratch_types=[
            pltpu.VMEM((2, _TK), jnp.int32),
            pltpu.VMEM((256,), jnp.int32),
            pltpu.VMEM((16,), jnp.int32),
        ],
        compiler_params=pltpu.CompilerParams(needs_layout_passes=False),
    )(sbits)


_NP = _H // 2


def _attn_body(q_ref, k_ref, v_ref, s_ref, t_ref, wo_ref, b_ref, o_ref):
    j = pl.program_id(1)
    thr = t_ref[0][:, 0:1]
    mask = s_ref[0] >= thr
    qp = q_ref[0, 0]
    k2 = k_ref[0]
    att = lax.dot_general(
        qp, k2, (((1,), (1,)), ((), ())), preferred_element_type=jnp.float32
    ) * (1.0 / math.sqrt(_DH))
    mask2 = jnp.concatenate([mask, mask], axis=0)
    am = jnp.where(mask2, att, -1e30)
    m = jnp.max(am, axis=-1, keepdims=True)
    e = jnp.exp(am - m)
    ssum = jnp.sum(e, axis=-1, keepdims=True)
    aw = e / ssum
    v2 = v_ref[0]
    pv = jnp.dot(aw, v2, preferred_element_type=jnp.float32)
    row_id = lax.broadcasted_iota(jnp.int32, (2 * _TQ, 128), 0)
    lane_id = lax.broadcasted_iota(jnp.int32, (2 * _TQ, 128), 1)
    keep = (row_id < _TQ) == (lane_id < _DH)
    pvc = jnp.where(keep, pv, 0.0)
    c2 = jnp.dot(pvc, wo_ref[...], preferred_element_type=jnp.float32)
    contrib = c2[0:_TQ, :] + c2[_TQ : 2 * _TQ, :]

    @pl.when(j == 0)
    def _():
        o_ref[0] = contrib + b_ref[...]

    @pl.when(j != 0)
    def _():
        o_ref[0] += contrib


def _attn(q_pair, kv, scores, thr_b, woT, bo):
    grid = (_B, _NP)
    return pl.pallas_call(
        _attn_body,
        grid=grid,
        in_specs=[
            pl.BlockSpec((1, 1, 2 * _TQ, 128), lambda b, j: (b, j, 0, 0)),
            pl.BlockSpec((1, _TK, 128), lambda b, j: (b, 0, j)),
            pl.BlockSpec((1, _TK, 128), lambda b, j: (b, 0, j + _NP)),
            pl.BlockSpec((1, _TQ, _TK), lambda b, j: (b, 0, 0)),
            pl.BlockSpec((1, _TQ, 128), lambda b, j: (b, 0, 0)),
            pl.BlockSpec((128, _C), lambda b, j: (j, 0)),
            pl.BlockSpec((1, _C), lambda b, j: (0, 0)),
        ],
        out_specs=pl.BlockSpec((1, _TQ, _C), lambda b, j: (b, 0, 0)),
        out_shape=jax.ShapeDtypeStruct((_B, _TQ, _C), jnp.float32),
        compiler_params=pltpu.CompilerParams(
            dimension_semantics=("parallel", "arbitrary"),
        ),
    )(q_pair, kv, kv, scores, thr_b, woT, bo)


def kernel(query, key, value, Wqi, Wki, head_weights, Wqkv, bqkv, Wout, bout):
    del value
    w_q = jax.nn.softmax(head_weights).astype(jnp.bfloat16).astype(jnp.float32)

    qf = query.reshape(_B * _TQ, _C)
    iq, Q = _proj_q(qf, Wqi.T, Wqkv[:_C].T, bqkv[:_C].reshape(1, _C))
    iq_r = iq.reshape(_B, _TQ, _HI, _DI).transpose(0, 2, 1, 3)
    q_h = Q.reshape(_B, _TQ, _H, _DH).transpose(0, 2, 1, 3)
    qp = jnp.zeros((_B, _NP, 2, _TQ, 2, _DH), jnp.float32)
    qpr = q_h.reshape(_B, _NP, 2, _TQ, _DH)
    qp = qp.at[:, :, 0, :, 0].set(qpr[:, :, 0])
    qp = qp.at[:, :, 1, :, 1].set(qpr[:, :, 1])
    q_pair = qp.reshape(_B, _NP, 2 * _TQ, 2 * _DH)

    kv, scores = _proj_k(
        key, Wqkv[_C:].T, bqkv[_C:].reshape(1, 2 * _C), Wki.T, iq_r, w_q
    )

    sbits = lax.bitcast_convert_type(scores.reshape(_B * _TQ, _TK), jnp.int32)
    tbits = _topk_sc(sbits)
    thr = lax.bitcast_convert_type(tbits, jnp.float32)[:, :2].reshape(_B, _TQ, 1)
    thr_b = jnp.broadcast_to(thr, (_B, _TQ, 128))

    out = _attn(q_pair, kv, scores, thr_b, Wout.T, bout.reshape(1, _C))
    return out


# speedup vs baseline: 3.6160x; 1.0071x over previous
import functools
import math

import jax
import jax.numpy as jnp
from jax import lax
from jax.experimental import pallas as pl
from jax.experimental.pallas import tpu as pltpu
from jax.experimental.pallas import tpu_sc as plsc

_B, _TQ, _TK, _C = 4, 16, 8192, 1024
_H, _DH = 16, 64
_HI, _DI = 32, 64
_TOPK = 1024

_BM = 512


def _proj_q_body(q_ref, wqiT_ref, wQT_ref, bq_ref, iq_ref, qo_ref):
    q = q_ref[...]
    iq_ref[...] = jnp.dot(q, wqiT_ref[...], preferred_element_type=jnp.float32)
    qo_ref[...] = (
        jnp.dot(q, wQT_ref[...], preferred_element_type=jnp.float32) + bq_ref[...]
    )


def _proj_q(qf, wqiT, wQT, bq):
    return pl.pallas_call(
        _proj_q_body,
        out_shape=(
            jax.ShapeDtypeStruct((_B * _TQ, _HI * _DI), jnp.float32),
            jax.ShapeDtypeStruct((_B * _TQ, _C), jnp.float32),
        ),
    )(qf, wqiT, wQT, bq)


def _proj_k_body(x_ref, wkvT_ref, bkv_ref, wkiT_ref, iq_ref, w_ref, kv_ref, s_ref, sb_ref):
    x = x_ref[0]
    kv_ref[0] = (
        jnp.dot(x, wkvT_ref[...], preferred_element_type=jnp.float32) + bkv_ref[...]
    )
    ik = jnp.dot(x, wkiT_ref[...], preferred_element_type=jnp.float32)
    acc = jnp.zeros((_TQ, _BM), jnp.float32)
    for h in range(_HI):
        iq_h = iq_ref[0, h]
        ik_h = ik[:, h * _DI : (h + 1) * _DI]
        s = lax.dot_general(
            iq_h, ik_h, (((1,), (1,)), ((), ())), preferred_element_type=jnp.float32
        )
        v = jnp.maximum(s * (1.0 / math.sqrt(_DI)), 0.0)
        vq = v.astype(jnp.bfloat16).astype(jnp.float32)
        acc = acc + vq * w_ref[h]
    s_ref[0] = acc
    sb_ref[...] = lax.bitcast_convert_type(acc, jnp.int32)


def _proj_k(key, wkvT, bkv, wkiT, iq_r, w_q):
    grid = (_B, _TK // _BM)
    return pl.pallas_call(
        _proj_k_body,
        grid=grid,
        in_specs=[
            pl.BlockSpec((1, _BM, _C), lambda b, m: (b, m, 0)),
            pl.BlockSpec((_C, 2 * _C), lambda b, m: (0, 0)),
            pl.BlockSpec((1, 2 * _C), lambda b, m: (0, 0)),
            pl.BlockSpec((_C, _HI * _DI), lambda b, m: (0, 0)),
            pl.BlockSpec((1, _HI, _TQ, _DI), lambda b, m: (b, 0, 0, 0)),
            pl.BlockSpec(memory_space=pltpu.SMEM),
        ],
        out_specs=(
            pl.BlockSpec((1, _BM, 2 * _C), lambda b, m: (b, m, 0)),
            pl.BlockSpec((1, _TQ, _BM), lambda b, m: (b, 0, m)),
            pl.BlockSpec((_TQ, _BM), lambda b, m: (b, m)),
        ),
        out_shape=(
            jax.ShapeDtypeStruct((_B, _TK, 2 * _C), jnp.float32),
            jax.ShapeDtypeStruct((_B, _TQ, _TK), jnp.float32),
            jax.ShapeDtypeStruct((_B * _TQ, _TK), jnp.int32),
        ),
        compiler_params=pltpu.CompilerParams(
            dimension_semantics=("parallel", "parallel"),
        ),
    )(key, wkvT, bkv, wkiT, iq_r, w_q)


def _topk_sc_body(sbits_ref, out_ref, rows_v, hist_v, out_v):
    c = lax.axis_index("c")
    s = lax.axis_index("s")
    wid = c * 16 + s
    base = wid * 2
    pltpu.sync_copy(sbits_ref.at[pl.ds(base, 2)], rows_v)
    lanes = lax.iota(jnp.int32, 16)
    ones = jnp.ones((16,), jnp.int32)
    zeros16 = jnp.zeros((16,), jnp.int32)
    res = jnp.zeros((16,), jnp.int32)
    for r in range(2):
        prefix = jnp.int32(0)
        krem = jnp.int32(_TOPK)
        for sh, nb in ((23, 8), (15, 8), (7, 8), (0, 7)):
            mask_d = (1 << nb) - 1
            hb = sh + nb
            for cc in range(16):
                hist_v[pl.ds(cc * 16, 16)] = zeros16

            def body(i, carry, r=r, sh=sh, hb=hb, mask_d=mask_d, prefix=prefix):
                x = rows_v[r, pl.ds(i * 16, 16)]
                digit = lax.shift_right_logical(x, sh) & mask_d
                match = lax.shift_right_logical(x, hb) == prefix
                ridx = mask_d - digit
                plsc.addupdate_scatter(hist_v, [ridx], ones, mask=match)
                return carry

            lax.fori_loop(0, _TK // 16, body, 0, unroll=8)

            tot = jnp.int32(0)
            found = jnp.bool_(False)
            digit_sel = jnp.int32(0)
            above_sel = jnp.int32(0)
            for cc in range(16):
                hv = hist_v[pl.ds(cc * 16, 16)]
                cs = plsc.cumsum(hv) + tot
                ge = cs >= krem
                has = jnp.any(ge)
                pc = plsc.cumsum(ge.astype(jnp.int32))
                fm = ge & (pc == 1)
                cum_at = jnp.sum(jnp.where(fm, cs, 0))
                h_at = jnp.sum(jnp.where(fm, hv, 0))
                ridx_at = jnp.sum(jnp.where(fm, lanes, 0)) + cc * 16
                take = has & (~found)
                digit_sel = jnp.where(take, mask_d - ridx_at, digit_sel)
                above_sel = jnp.where(take, cum_at - h_at, above_sel)
                found = found | has
                tot = tot + jnp.sum(hv)
            krem = krem - above_sel
            prefix = jnp.bitwise_or(lax.shift_left(prefix, nb), digit_sel)
        res = jnp.where(lanes == r, prefix, res)
    out_v[...] = res
    pltpu.sync_copy(out_v, out_ref.at[wid])


def _topk_sc(sbits):
    mesh = plsc.VectorSubcoreMesh(core_axis_name="c", subcore_axis_name="s")
    return pl.kernel(
        _topk_sc_body,
        out_type=jax.ShapeDtypeStruct((32, 16), jnp.int32),
        mesh=mesh,
        scratch_types=[
            pltpu.VMEM((2, _TK), jnp.int32),
            pltpu.VMEM((256,), jnp.int32),
            pltpu.VMEM((16,), jnp.int32),
        ],
        compiler_params=pltpu.CompilerParams(needs_layout_passes=False),
    )(sbits)


_NP = _H // 2


def _attn_body(q_ref, k_ref, v_ref, s_ref, t_ref, wo_ref, b_ref, o_ref):
    j = pl.program_id(1)
    thr = t_ref[0][:, 0:1]
    mask = s_ref[0] >= thr
    z = jnp.zeros((_TQ, _DH), jnp.float32)
    qp = jnp.concatenate(
        [
            jnp.concatenate([q_ref[0, 0], z], axis=1),
            jnp.concatenate([z, q_ref[0, 1]], axis=1),
        ],
        axis=0,
    )
    k2 = k_ref[0]
    att = lax.dot_general(
        qp, k2, (((1,), (1,)), ((), ())), preferred_element_type=jnp.float32
    ) * (1.0 / math.sqrt(_DH))
    mask2 = jnp.concatenate([mask, mask], axis=0)
    am = jnp.where(mask2, att, -1e30)
    m = jnp.max(am, axis=-1, keepdims=True)
    e = jnp.exp(am - m)
    ssum = jnp.sum(e, axis=-1, keepdims=True)
    aw = e / ssum
    v2 = v_ref[0]
    pv = jnp.dot(aw, v2, preferred_element_type=jnp.float32)
    row_id = lax.broadcasted_iota(jnp.int32, (2 * _TQ, 128), 0)
    lane_id = lax.broadcasted_iota(jnp.int32, (2 * _TQ, 128), 1)
    keep = (row_id < _TQ) == (lane_id < _DH)
    pvc = jnp.where(keep, pv, 0.0)
    c2 = jnp.dot(pvc, wo_ref[...], preferred_element_type=jnp.float32)
    contrib = c2[0:_TQ, :] + c2[_TQ : 2 * _TQ, :]

    @pl.when(j == 0)
    def _():
        o_ref[0] = contrib + b_ref[...]

    @pl.when(j != 0)
    def _():
        o_ref[0] += contrib


def _attn(q_h, kv, scores, thr_b, woT, bo):
    grid = (_B, _NP)
    return pl.pallas_call(
        _attn_body,
        grid=grid,
        in_specs=[
            pl.BlockSpec((1, 2, _TQ, _DH), lambda b, j: (b, j, 0, 0)),
            pl.BlockSpec((1, _TK, 128), lambda b, j: (b, 0, j)),
            pl.BlockSpec((1, _TK, 128), lambda b, j: (b, 0, j + _NP)),
            pl.BlockSpec((1, _TQ, _TK), lambda b, j: (b, 0, 0)),
            pl.BlockSpec((1, _TQ, 128), lambda b, j: (b, 0, 0)),
            pl.BlockSpec((128, _C), lambda b, j: (j, 0)),
            pl.BlockSpec((1, _C), lambda b, j: (0, 0)),
        ],
        out_specs=pl.BlockSpec((1, _TQ, _C), lambda b, j: (b, 0, 0)),
        out_shape=jax.ShapeDtypeStruct((_B, _TQ, _C), jnp.float32),
        compiler_params=pltpu.CompilerParams(
            dimension_semantics=("parallel", "arbitrary"),
        ),
    )(q_h, kv, kv, scores, thr_b, woT, bo)


def kernel(query, key, value, Wqi, Wki, head_weights, Wqkv, bqkv, Wout, bout):
    del value
    w_q = jax.nn.softmax(head_weights).astype(jnp.bfloat16).astype(jnp.float32)

    qf = query.reshape(_B * _TQ, _C)
    iq, Q = _proj_q(qf, Wqi.T, Wqkv[:_C].T, bqkv[:_C].reshape(1, _C))
    iq_r = iq.reshape(_B, _TQ, _HI, _DI).transpose(0, 2, 1, 3)
    q_h = Q.reshape(_B, _TQ, _H, _DH).transpose(0, 2, 1, 3)

    kv, scores, sbits = _proj_k(
        key, Wqkv[_C:].T, bqkv[_C:].reshape(1, 2 * _C), Wki.T, iq_r, w_q
    )

    tbits = _topk_sc(sbits)
    thr = lax.bitcast_convert_type(tbits, jnp.float32)[:, :2].reshape(_B, _TQ, 1)
    thr_b = jnp.broadcast_to(thr, (_B, _TQ, 128))

    out = _attn(q_h, kv, scores, thr_b, Wout.T, bout.reshape(1, _C))
    return out


# speedup vs baseline: 4.0116x; 1.1094x over previous
import functools
import math

import jax
import jax.numpy as jnp
from jax import lax
from jax.experimental import pallas as pl
from jax.experimental.pallas import tpu as pltpu
from jax.experimental.pallas import tpu_sc as plsc

_B, _TQ, _TK, _C = 4, 16, 8192, 1024
_H, _DH = 16, 64
_HI, _DI = 32, 64
_TOPK = 1024

_BM = 1024
_HP = 4


def _proj_q_body(q_ref, wqiT_ref, wQT_ref, bq_ref, iq_ref, qo_ref):
    q = q_ref[...]
    iq_ref[...] = jnp.dot(q, wqiT_ref[...], preferred_element_type=jnp.float32)
    qo_ref[...] = (
        jnp.dot(q, wQT_ref[...], preferred_element_type=jnp.float32) + bq_ref[...]
    )


def _proj_q(qf, wqiT, wQT, bq):
    return pl.pallas_call(
        _proj_q_body,
        out_shape=(
            jax.ShapeDtypeStruct((_B * _TQ, _HI * _DI), jnp.float32),
            jax.ShapeDtypeStruct((_B * _TQ, _C), jnp.float32),
        ),
    )(qf, wqiT, wQT, bq)


def _proj_k_body(x_ref, wkvT_ref, bkv_ref, wkiT_ref, iq_ref, w_ref, kv_ref, s_ref, sb_ref):
    x = x_ref[0]
    kv_ref[0] = (
        jnp.dot(x, wkvT_ref[...], preferred_element_type=jnp.float32) + bkv_ref[...]
    ).astype(jnp.bfloat16)
    ik = jnp.dot(x, wkiT_ref[...], preferred_element_type=jnp.float32)
    acc = jnp.zeros((_TQ, _BM), jnp.float32)
    for g in range(_HI // _HP):
        iq_g = iq_ref[0, g]
        ik_g = ik[:, g * _HP * _DI : (g + 1) * _HP * _DI]
        sp = lax.dot_general(
            iq_g, ik_g, (((1,), (1,)), ((), ())), preferred_element_type=jnp.float32
        )
        for j in range(_HP):
            s = sp[j * _TQ : (j + 1) * _TQ]
            v = jnp.maximum(s * (1.0 / math.sqrt(_DI)), 0.0)
            vq = v.astype(jnp.bfloat16).astype(jnp.float32)
            acc = acc + vq * w_ref[g * _HP + j]
    s_ref[0] = acc
    sb_ref[...] = lax.bitcast_convert_type(acc, jnp.int32)


def _proj_k(key, wkvT, bkv, wkiT, iq_r, w_q):
    grid = (_B, _TK // _BM)
    return pl.pallas_call(
        _proj_k_body,
        grid=grid,
        in_specs=[
            pl.BlockSpec((1, _BM, _C), lambda b, m: (b, m, 0)),
            pl.BlockSpec((_C, 2 * _C), lambda b, m: (0, 0)),
            pl.BlockSpec((1, 2 * _C), lambda b, m: (0, 0)),
            pl.BlockSpec((_C, _HI * _DI), lambda b, m: (0, 0)),
            pl.BlockSpec(
                (1, _HI // _HP, _HP * _TQ, _HP * _DI), lambda b, m: (b, 0, 0, 0)
            ),
            pl.BlockSpec(memory_space=pltpu.SMEM),
        ],
        out_specs=(
            pl.BlockSpec((1, _BM, 2 * _C), lambda b, m: (b, m, 0)),
            pl.BlockSpec((1, _TQ, _BM), lambda b, m: (b, 0, m)),
            pl.BlockSpec((_TQ, _BM), lambda b, m: (b, m)),
        ),
        out_shape=(
            jax.ShapeDtypeStruct((_B, _TK, 2 * _C), jnp.bfloat16),
            jax.ShapeDtypeStruct((_B, _TQ, _TK), jnp.float32),
            jax.ShapeDtypeStruct((_B * _TQ, _TK), jnp.int32),
        ),
        compiler_params=pltpu.CompilerParams(
            dimension_semantics=("parallel", "parallel"),
        ),
    )(key, wkvT, bkv, wkiT, iq_r, w_q)


def _topk_sc_body(sbits_ref, out_ref, rows_v, hist_v, out_v):
    c = lax.axis_index("c")
    s = lax.axis_index("s")
    wid = c * 16 + s
    base = wid * 2
    pltpu.sync_copy(sbits_ref.at[pl.ds(base, 2)], rows_v)
    lanes = lax.iota(jnp.int32, 16)
    ones = jnp.ones((16,), jnp.int32)
    zeros16 = jnp.zeros((16,), jnp.int32)
    res = jnp.zeros((16,), jnp.int32)
    for r in range(2):
        prefix = jnp.int32(0)
        krem = jnp.int32(_TOPK)
        for sh, nb in ((23, 8), (15, 8), (7, 8), (0, 7)):
            mask_d = (1 << nb) - 1
            hb = sh + nb
            for cc in range(16):
                hist_v[pl.ds(cc * 16, 16)] = zeros16

            def body(i, carry, r=r, sh=sh, hb=hb, mask_d=mask_d, prefix=prefix):
                x = rows_v[r, pl.ds(i * 16, 16)]
                digit = lax.shift_right_logical(x, sh) & mask_d
                match = lax.shift_right_logical(x, hb) == prefix
                ridx = mask_d - digit
                plsc.addupdate_scatter(hist_v, [ridx], ones, mask=match)
                return carry

            lax.fori_loop(0, _TK // 16, body, 0, unroll=8)

            tot = jnp.int32(0)
            found = jnp.bool_(False)
            digit_sel = jnp.int32(0)
            above_sel = jnp.int32(0)
            for cc in range(16):
                hv = hist_v[pl.ds(cc * 16, 16)]
                cs = plsc.cumsum(hv) + tot
                ge = cs >= krem
                has = jnp.any(ge)
                pc = plsc.cumsum(ge.astype(jnp.int32))
                fm = ge & (pc == 1)
                cum_at = jnp.sum(jnp.where(fm, cs, 0))
                h_at = jnp.sum(jnp.where(fm, hv, 0))
                ridx_at = jnp.sum(jnp.where(fm, lanes, 0)) + cc * 16
                take = has & (~found)
                digit_sel = jnp.where(take, mask_d - ridx_at, digit_sel)
                above_sel = jnp.where(take, cum_at - h_at, above_sel)
                found = found | has
                tot = tot + jnp.sum(hv)
            krem = krem - above_sel
            prefix = jnp.bitwise_or(lax.shift_left(prefix, nb), digit_sel)
        res = jnp.where(lanes == r, prefix, res)
    out_v[...] = res
    pltpu.sync_copy(out_v, out_ref.at[wid])


def _topk_sc(sbits):
    mesh = plsc.VectorSubcoreMesh(core_axis_name="c", subcore_axis_name="s")
    return pl.kernel(
        _topk_sc_body,
        out_type=jax.ShapeDtypeStruct((32, 16), jnp.int32),
        mesh=mesh,
        scratch_types=[
            pltpu.VMEM((2, _TK), jnp.int32),
            pltpu.VMEM((256,), jnp.int32),
            pltpu.VMEM((16,), jnp.int32),
        ],
        compiler_params=pltpu.CompilerParams(needs_layout_passes=False),
    )(sbits)


_NP = _H // 2


def _attn_body(q_ref, k_ref, v_ref, s_ref, t_ref, wo_ref, b_ref, o_ref):
    j = pl.program_id(1)
    thr = t_ref[0][:, 0:1]
    mask = s_ref[0] >= thr
    z = jnp.zeros((_TQ, _DH), jnp.float32)
    qp = jnp.concatenate(
        [
            jnp.concatenate([q_ref[0, 0], z], axis=1),
            jnp.concatenate([z, q_ref[0, 1]], axis=1),
        ],
        axis=0,
    ).astype(jnp.bfloat16)
    k2 = k_ref[0]
    att = lax.dot_general(
        qp, k2, (((1,), (1,)), ((), ())), preferred_element_type=jnp.float32
    ) * (1.0 / math.sqrt(_DH))
    mask2 = jnp.concatenate([mask, mask], axis=0)
    am = jnp.where(mask2, att, -1e30)
    m = jnp.max(am, axis=-1, keepdims=True)
    e = jnp.exp(am - m)
    ssum = jnp.sum(e, axis=-1, keepdims=True)
    aw = (e / ssum).astype(jnp.bfloat16)
    v2 = v_ref[0]
    pv = jnp.dot(aw, v2, preferred_element_type=jnp.float32)
    row_id = lax.broadcasted_iota(jnp.int32, (2 * _TQ, 128), 0)
    lane_id = lax.broadcasted_iota(jnp.int32, (2 * _TQ, 128), 1)
    keep = (row_id < _TQ) == (lane_id < _DH)
    pvc = jnp.where(keep, pv, 0.0)
    c2 = jnp.dot(pvc, wo_ref[...], preferred_element_type=jnp.float32)
    contrib = c2[0:_TQ, :] + c2[_TQ : 2 * _TQ, :]

    @pl.when(j == 0)
    def _():
        o_ref[0] = contrib + b_ref[...]

    @pl.when(j != 0)
    def _():
        o_ref[0] += contrib


def _attn(q_h, kv, scores, thr_b, woT, bo):
    grid = (_B, _NP)
    return pl.pallas_call(
        _attn_body,
        grid=grid,
        in_specs=[
            pl.BlockSpec((1, 2, _TQ, _DH), lambda b, j: (b, j, 0, 0)),
            pl.BlockSpec((1, _TK, 128), lambda b, j: (b, 0, j)),
            pl.BlockSpec((1, _TK, 128), lambda b, j: (b, 0, j + _NP)),
            pl.BlockSpec((1, _TQ, _TK), lambda b, j: (b, 0, 0)),
            pl.BlockSpec((1, _TQ, 128), lambda b, j: (b, 0, 0)),
            pl.BlockSpec((128, _C), lambda b, j: (j, 0)),
            pl.BlockSpec((1, _C), lambda b, j: (0, 0)),
        ],
        out_specs=pl.BlockSpec((1, _TQ, _C), lambda b, j: (b, 0, 0)),
        out_shape=jax.ShapeDtypeStruct((_B, _TQ, _C), jnp.float32),
        compiler_params=pltpu.CompilerParams(
            dimension_semantics=("parallel", "arbitrary"),
        ),
    )(q_h, kv, kv, scores, thr_b, woT, bo)


def kernel(query, key, value, Wqi, Wki, head_weights, Wqkv, bqkv, Wout, bout):
    del value
    w_q = jax.nn.softmax(head_weights).astype(jnp.bfloat16).astype(jnp.float32)

    qf = query.reshape(_B * _TQ, _C)
    iq, Q = _proj_q(qf, Wqi.T, Wqkv[:_C].T, bqkv[:_C].reshape(1, _C))
    iq_r = iq.reshape(_B, _TQ, _HI, _DI).transpose(0, 2, 1, 3)
    q_h = Q.reshape(_B, _TQ, _H, _DH).transpose(0, 2, 1, 3)

    ng = _HI // _HP
    iqg = iq_r.reshape(_B, ng, _HP, _TQ, _DI)
    iqp = jnp.zeros((_B, ng, _HP, _TQ, _HP, _DI), jnp.float32)
    for j in range(_HP):
        iqp = iqp.at[:, :, j, :, j].set(iqg[:, :, j])
    iq_pack = iqp.reshape(_B, ng, _HP * _TQ, _HP * _DI)

    kv, scores, sbits = _proj_k(
        key, Wqkv[_C:].T, bqkv[_C:].reshape(1, 2 * _C), Wki.T, iq_pack, w_q
    )

    tbits = _topk_sc(sbits)
    thr = lax.bitcast_convert_type(tbits, jnp.float32)[:, :2].reshape(_B, _TQ, 1)
    thr_b = jnp.broadcast_to(thr, (_B, _TQ, 128))

    out = _attn(q_h, kv, scores, thr_b, Wout.T, bout.reshape(1, _C))
    return out


# speedup vs baseline: 4.3381x; 1.0814x over previous
import functools
import math

import jax
import jax.numpy as jnp
from jax import lax
from jax.experimental import pallas as pl
from jax.experimental.pallas import tpu as pltpu
from jax.experimental.pallas import tpu_sc as plsc

_B, _TQ, _TK, _C = 4, 16, 8192, 1024
_H, _DH = 16, 64
_HI, _DI = 32, 64
_TOPK = 1024

_BM = 1024
_HP = 4


def _proj_q_body(q_ref, wqiT_ref, wQT_ref, bq_ref, iq_ref, qo_ref):
    q = q_ref[...]
    iq_ref[...] = jnp.dot(q, wqiT_ref[...], preferred_element_type=jnp.float32)
    qo_ref[...] = (
        jnp.dot(q, wQT_ref[...], preferred_element_type=jnp.float32) + bq_ref[...]
    )


def _proj_q(qf, wqiT, wQT, bq):
    return pl.pallas_call(
        _proj_q_body,
        out_shape=(
            jax.ShapeDtypeStruct((_B * _TQ, _HI * _DI), jnp.float32),
            jax.ShapeDtypeStruct((_B * _TQ, _C), jnp.float32),
        ),
    )(qf, wqiT, wQT, bq)


def _scores_body(x_ref, wkiT_ref, iq_ref, w_ref, s_ref, sb_ref):
    x = x_ref[0]
    ik = jnp.dot(x, wkiT_ref[...], preferred_element_type=jnp.float32)
    iqb = iq_ref[0]
    gdim = _HP * _DI
    row_id = lax.broadcasted_iota(jnp.int32, (_HP * _TQ, gdim), 0)
    lane_id = lax.broadcasted_iota(jnp.int32, (_HP * _TQ, gdim), 1)
    diag = (row_id // _TQ) == (lane_id // _DI)
    acc = jnp.zeros((_TQ, _BM), jnp.float32)
    for g in range(_HI // _HP):
        sl = iqb[:, g * gdim : (g + 1) * gdim]
        tile = jnp.concatenate([sl] * _HP, axis=0)
        iq_g = jnp.where(diag, tile, 0.0)
        ik_g = ik[:, g * gdim : (g + 1) * gdim]
        sp = lax.dot_general(
            iq_g, ik_g, (((1,), (1,)), ((), ())), preferred_element_type=jnp.float32
        )
        for j in range(_HP):
            s = sp[j * _TQ : (j + 1) * _TQ]
            v = jnp.maximum(s * (1.0 / math.sqrt(_DI)), 0.0)
            vq = v.astype(jnp.bfloat16).astype(jnp.float32)
            acc = acc + vq * w_ref[g * _HP + j]
    s_ref[0] = acc
    sb_ref[...] = lax.bitcast_convert_type(acc, jnp.int32)


def _scores_k(key, wkiT, iq, w_q):
    grid = (_B, _TK // _BM)
    return pl.pallas_call(
        _scores_body,
        grid=grid,
        in_specs=[
            pl.BlockSpec((1, _BM, _C), lambda b, m: (b, m, 0)),
            pl.BlockSpec((_C, _HI * _DI), lambda b, m: (0, 0)),
            pl.BlockSpec((1, _TQ, _HI * _DI), lambda b, m: (b, 0, 0)),
            pl.BlockSpec(memory_space=pltpu.SMEM),
        ],
        out_specs=(
            pl.BlockSpec((1, _TQ, _BM), lambda b, m: (b, 0, m)),
            pl.BlockSpec((_TQ, _BM), lambda b, m: (b, m)),
        ),
        out_shape=(
            jax.ShapeDtypeStruct((_B, _TQ, _TK), jnp.float32),
            jax.ShapeDtypeStruct((_B * _TQ, _TK), jnp.int32),
        ),
        compiler_params=pltpu.CompilerParams(
            dimension_semantics=("parallel", "parallel"),
        ),
    )(key, wkiT, iq, w_q)


def _kv_body(x_ref, wkvT_ref, bkv_ref, kv_ref):
    x = x_ref[0]
    kv_ref[0] = (
        jnp.dot(x, wkvT_ref[...], preferred_element_type=jnp.float32) + bkv_ref[...]
    ).astype(jnp.bfloat16)


def _kv_k(key, wkvT, bkv):
    grid = (_B, _TK // _BM)
    return pl.pallas_call(
        _kv_body,
        grid=grid,
        in_specs=[
            pl.BlockSpec((1, _BM, _C), lambda b, m: (b, m, 0)),
            pl.BlockSpec((_C, 2 * _C), lambda b, m: (0, 0)),
            pl.BlockSpec((1, 2 * _C), lambda b, m: (0, 0)),
        ],
        out_specs=pl.BlockSpec((1, _BM, 2 * _C), lambda b, m: (b, m, 0)),
        out_shape=jax.ShapeDtypeStruct((_B, _TK, 2 * _C), jnp.bfloat16),
        compiler_params=pltpu.CompilerParams(
            dimension_semantics=("parallel", "parallel"),
        ),
    )(key, wkvT, bkv)


def _topk_sc_body(sbits_ref, out_ref, rows_v, hist_v, out_v):
    c = lax.axis_index("c")
    s = lax.axis_index("s")
    wid = c * 16 + s
    base = wid * 2
    pltpu.sync_copy(sbits_ref.at[pl.ds(base, 2)], rows_v)
    lanes = lax.iota(jnp.int32, 16)
    ones = jnp.ones((16,), jnp.int32)
    zeros16 = jnp.zeros((16,), jnp.int32)
    res = jnp.zeros((16,), jnp.int32)
    for r in range(2):
        prefix = jnp.int32(0)
        krem = jnp.int32(_TOPK)
        for sh, nb in ((23, 8), (15, 8), (7, 8), (0, 7)):
            mask_d = (1 << nb) - 1
            hb = sh + nb
            for cc in range(16):
                hist_v[pl.ds(cc * 16, 16)] = zeros16

            def body(i, carry, r=r, sh=sh, hb=hb, mask_d=mask_d, prefix=prefix):
                x = rows_v[r, pl.ds(i * 16, 16)]
                digit = lax.shift_right_logical(x, sh) & mask_d
                match = lax.shift_right_logical(x, hb) == prefix
                ridx = mask_d - digit
                plsc.addupdate_scatter(hist_v, [ridx], ones, mask=match)
                return carry

            lax.fori_loop(0, _TK // 16, body, 0, unroll=8)

            tot = jnp.int32(0)
            found = jnp.bool_(False)
            digit_sel = jnp.int32(0)
            above_sel = jnp.int32(0)
            for cc in range(16):
                hv = hist_v[pl.ds(cc * 16, 16)]
                cs = plsc.cumsum(hv) + tot
                ge = cs >= krem
                has = jnp.any(ge)
                pc = plsc.cumsum(ge.astype(jnp.int32))
                fm = ge & (pc == 1)
                cum_at = jnp.sum(jnp.where(fm, cs, 0))
                h_at = jnp.sum(jnp.where(fm, hv, 0))
                ridx_at = jnp.sum(jnp.where(fm, lanes, 0)) + cc * 16
                take = has & (~found)
                digit_sel = jnp.where(take, mask_d - ridx_at, digit_sel)
                above_sel = jnp.where(take, cum_at - h_at, above_sel)
                found = found | has
                tot = tot + jnp.sum(hv)
            krem = krem - above_sel
            prefix = jnp.bitwise_or(lax.shift_left(prefix, nb), digit_sel)
        res = jnp.where(lanes == r, prefix, res)
    out_v[...] = res
    pltpu.sync_copy(out_v, out_ref.at[wid])


def _topk_sc(sbits):
    mesh = plsc.VectorSubcoreMesh(core_axis_name="c", subcore_axis_name="s")
    return pl.kernel(
        _topk_sc_body,
        out_type=jax.ShapeDtypeStruct((32, 16), jnp.int32),
        mesh=mesh,
        scratch_types=[
            pltpu.VMEM((2, _TK), jnp.int32),
            pltpu.VMEM((256,), jnp.int32),
            pltpu.VMEM((16,), jnp.int32),
        ],
        compiler_params=pltpu.CompilerParams(needs_layout_passes=False),
    )(sbits)


_NP = _H // 2


def _attn_body(q_ref, k_ref, v_ref, s_ref, t_ref, wo_ref, b_ref, o_ref):
    j = pl.program_id(1)
    thr = t_ref[0][:, 0:1]
    mask = s_ref[0] >= thr
    q2 = q_ref[0]
    tile = jnp.concatenate([q2, q2], axis=0)
    row_id = lax.broadcasted_iota(jnp.int32, (2 * _TQ, 128), 0)
    lane_id = lax.broadcasted_iota(jnp.int32, (2 * _TQ, 128), 1)
    qp = jnp.where((row_id // _TQ) == (lane_id // _DH), tile, 0.0).astype(
        jnp.bfloat16
    )
    k2 = k_ref[0]
    att = lax.dot_general(
        qp, k2, (((1,), (1,)), ((), ())), preferred_element_type=jnp.float32
    ) * (1.0 / math.sqrt(_DH))
    mask2 = jnp.concatenate([mask, mask], axis=0)
    am = jnp.where(mask2, att, -1e30)
    m = jnp.max(am, axis=-1, keepdims=True)
    e = jnp.exp(am - m)
    ssum = jnp.sum(e, axis=-1, keepdims=True)
    aw = (e / ssum).astype(jnp.bfloat16)
    v2 = v_ref[0]
    pv = jnp.dot(aw, v2, preferred_element_type=jnp.float32)
    row_id = lax.broadcasted_iota(jnp.int32, (2 * _TQ, 128), 0)
    lane_id = lax.broadcasted_iota(jnp.int32, (2 * _TQ, 128), 1)
    keep = (row_id < _TQ) == (lane_id < _DH)
    pvc = jnp.where(keep, pv, 0.0)
    c2 = jnp.dot(pvc, wo_ref[...], preferred_element_type=jnp.float32)
    contrib = c2[0:_TQ, :] + c2[_TQ : 2 * _TQ, :]

    @pl.when(j == 0)
    def _():
        o_ref[0] = contrib + b_ref[...]

    @pl.when(j != 0)
    def _():
        o_ref[0] += contrib


def _attn(q_h, kv, scores, thr_b, woT, bo):
    grid = (_B, _NP)
    return pl.pallas_call(
        _attn_body,
        grid=grid,
        in_specs=[
            pl.BlockSpec((1, _TQ, 128), lambda b, j: (b, 0, j)),
            pl.BlockSpec((1, _TK, 128), lambda b, j: (b, 0, j)),
            pl.BlockSpec((1, _TK, 128), lambda b, j: (b, 0, j + _NP)),
            pl.BlockSpec((1, _TQ, _TK), lambda b, j: (b, 0, 0)),
            pl.BlockSpec((1, _TQ, 128), lambda b, j: (b, 0, 0)),
            pl.BlockSpec((128, _C), lambda b, j: (j, 0)),
            pl.BlockSpec((1, _C), lambda b, j: (0, 0)),
        ],
        out_specs=pl.BlockSpec((1, _TQ, _C), lambda b, j: (b, 0, 0)),
        out_shape=jax.ShapeDtypeStruct((_B, _TQ, _C), jnp.float32),
        compiler_params=pltpu.CompilerParams(
            dimension_semantics=("parallel", "arbitrary"),
        ),
    )(q_h, kv, kv, scores, thr_b, woT, bo)


def kernel(query, key, value, Wqi, Wki, head_weights, Wqkv, bqkv, Wout, bout):
    del value
    w_q = jax.nn.softmax(head_weights).astype(jnp.bfloat16).astype(jnp.float32)

    qf = query.reshape(_B * _TQ, _C)
    iq, Q = _proj_q(qf, Wqi.T, Wqkv[:_C].T, bqkv[:_C].reshape(1, _C))
    scores, sbits = _scores_k(key, Wki.T, iq.reshape(_B, _TQ, _HI * _DI), w_q)

    tbits = _topk_sc(sbits)
    thr = lax.bitcast_convert_type(tbits, jnp.float32)[:, :2].reshape(_B, _TQ, 1)
    thr_b = jnp.broadcast_to(thr, (_B, _TQ, 128))

    kv = _kv_k(key, Wqkv[_C:].T, bqkv[_C:].reshape(1, 2 * _C))

    out = _attn(
        Q.reshape(_B, _TQ, _C), kv, scores, thr_b, Wout.T, bout.reshape(1, _C)
    )
    return out


# speedup vs baseline: 4.4235x; 1.0197x over previous
import functools
import math

import jax
import jax.numpy as jnp
from jax import lax
from jax.experimental import pallas as pl
from jax.experimental.pallas import tpu as pltpu
from jax.experimental.pallas import tpu_sc as plsc

_B, _TQ, _TK, _C = 4, 16, 8192, 1024
_H, _DH = 16, 64
_HI, _DI = 32, 64
_TOPK = 1024

_BM = 1024
_HP = 4


_DNT = (((1,), (1,)), ((), ()))


def _proj_q_body(q_ref, wqi_ref, wQ_ref, bq_ref, iq_ref, qo_ref):
    q = q_ref[...]
    iq_ref[...] = lax.dot_general(
        q, wqi_ref[...], _DNT, preferred_element_type=jnp.float32
    )
    qo_ref[...] = (
        lax.dot_general(q, wQ_ref[...], _DNT, preferred_element_type=jnp.float32)
        + bq_ref[...]
    )


def _proj_q(qf, wqi, wQ, bq):
    return pl.pallas_call(
        _proj_q_body,
        out_shape=(
            jax.ShapeDtypeStruct((_B * _TQ, _HI * _DI), jnp.float32),
            jax.ShapeDtypeStruct((_B * _TQ, _C), jnp.float32),
        ),
    )(qf, wqi, wQ, bq)


def _scores_body(x_ref, wki_ref, iq_ref, w_ref, s_ref, sb_ref):
    x = x_ref[0]
    ik = lax.dot_general(
        x, wki_ref[...], _DNT, preferred_element_type=jnp.float32
    )
    iqb = iq_ref[0]
    gdim = _HP * _DI
    row_id = lax.broadcasted_iota(jnp.int32, (_HP * _TQ, gdim), 0)
    lane_id = lax.broadcasted_iota(jnp.int32, (_HP * _TQ, gdim), 1)
    diag = (row_id // _TQ) == (lane_id // _DI)
    acc = jnp.zeros((_TQ, _BM), jnp.float32)
    for g in range(_HI // _HP):
        sl = iqb[:, g * gdim : (g + 1) * gdim]
        tile = jnp.concatenate([sl] * _HP, axis=0)
        iq_g = jnp.where(diag, tile, 0.0)
        ik_g = ik[:, g * gdim : (g + 1) * gdim]
        sp = lax.dot_general(
            iq_g, ik_g, (((1,), (1,)), ((), ())), preferred_element_type=jnp.float32
        )
        for j in range(_HP):
            s = sp[j * _TQ : (j + 1) * _TQ]
            v = jnp.maximum(s * (1.0 / math.sqrt(_DI)), 0.0)
            vq = v.astype(jnp.bfloat16).astype(jnp.float32)
            acc = acc + vq * w_ref[g * _HP + j]
    s_ref[0] = acc
    sb_ref[...] = lax.bitcast_convert_type(acc, jnp.int32)


def _scores_k(key, wki, iq, w_q):
    grid = (_B, _TK // _BM)
    return pl.pallas_call(
        _scores_body,
        grid=grid,
        in_specs=[
            pl.BlockSpec((1, _BM, _C), lambda b, m: (b, m, 0)),
            pl.BlockSpec((_HI * _DI, _C), lambda b, m: (0, 0)),
            pl.BlockSpec((1, _TQ, _HI * _DI), lambda b, m: (b, 0, 0)),
            pl.BlockSpec(memory_space=pltpu.SMEM),
        ],
        out_specs=(
            pl.BlockSpec((1, _TQ, _BM), lambda b, m: (b, 0, m)),
            pl.BlockSpec((_TQ, _BM), lambda b, m: (b, m)),
        ),
        out_shape=(
            jax.ShapeDtypeStruct((_B, _TQ, _TK), jnp.float32),
            jax.ShapeDtypeStruct((_B * _TQ, _TK), jnp.int32),
        ),
        compiler_params=pltpu.CompilerParams(
            dimension_semantics=("parallel", "parallel"),
        ),
    )(key, wki, iq, w_q)


def _kv_body(x_ref, wkv_ref, bkv_ref, kv_ref):
    x = x_ref[0]
    kv_ref[0] = (
        lax.dot_general(x, wkv_ref[...], _DNT, preferred_element_type=jnp.float32)
        + bkv_ref[...]
    ).astype(jnp.bfloat16)


def _kv_k(key, wkv, bkv):
    grid = (_B, _TK // _BM)
    return pl.pallas_call(
        _kv_body,
        grid=grid,
        in_specs=[
            pl.BlockSpec((1, _BM, _C), lambda b, m: (b, m, 0)),
            pl.BlockSpec((2 * _C, _C), lambda b, m: (0, 0)),
            pl.BlockSpec((1, 2 * _C), lambda b, m: (0, 0)),
        ],
        out_specs=pl.BlockSpec((1, _BM, 2 * _C), lambda b, m: (b, m, 0)),
        out_shape=jax.ShapeDtypeStruct((_B, _TK, 2 * _C), jnp.bfloat16),
        compiler_params=pltpu.CompilerParams(
            dimension_semantics=("parallel", "parallel"),
        ),
    )(key, wkv, bkv)


def _topk_sc_body(sbits_ref, out_ref, rows_v, hist_v, out_v):
    c = lax.axis_index("c")
    s = lax.axis_index("s")
    wid = c * 16 + s
    base = wid * 2
    pltpu.sync_copy(sbits_ref.at[pl.ds(base, 2)], rows_v)
    lanes = lax.iota(jnp.int32, 16)
    ones = jnp.ones((16,), jnp.int32)
    zeros16 = jnp.zeros((16,), jnp.int32)
    res = jnp.zeros((16,), jnp.int32)
    for r in range(2):
        prefix = jnp.int32(0)
        krem = jnp.int32(_TOPK)
        for sh, nb in ((23, 8), (15, 8), (7, 8), (0, 7)):
            mask_d = (1 << nb) - 1
            hb = sh + nb
            for cc in range(16):
                hist_v[pl.ds(cc * 16, 16)] = zeros16

            def body(i, carry, r=r, sh=sh, hb=hb, mask_d=mask_d, prefix=prefix):
                x = rows_v[r, pl.ds(i * 16, 16)]
                digit = lax.shift_right_logical(x, sh) & mask_d
                match = lax.shift_right_logical(x, hb) == prefix
                ridx = mask_d - digit
                plsc.addupdate_scatter(hist_v, [ridx], ones, mask=match)
                return carry

            lax.fori_loop(0, _TK // 16, body, 0, unroll=8)

            tot = jnp.int32(0)
            found = jnp.bool_(False)
            digit_sel = jnp.int32(0)
            above_sel = jnp.int32(0)
            for cc in range(16):
                hv = hist_v[pl.ds(cc * 16, 16)]
                cs = plsc.cumsum(hv) + tot
                ge = cs >= krem
                has = jnp.any(ge)
                pc = plsc.cumsum(ge.astype(jnp.int32))
                fm = ge & (pc == 1)
                cum_at = jnp.sum(jnp.where(fm, cs, 0))
                h_at = jnp.sum(jnp.where(fm, hv, 0))
                ridx_at = jnp.sum(jnp.where(fm, lanes, 0)) + cc * 16
                take = has & (~found)
                digit_sel = jnp.where(take, mask_d - ridx_at, digit_sel)
                above_sel = jnp.where(take, cum_at - h_at, above_sel)
                found = found | has
                tot = tot + jnp.sum(hv)
            krem = krem - above_sel
            prefix = jnp.bitwise_or(lax.shift_left(prefix, nb), digit_sel)
        res = jnp.where(lanes == r, prefix, res)
    out_v[...] = res
    pltpu.sync_copy(out_v, out_ref.at[wid])


def _topk_sc(sbits):
    mesh = plsc.VectorSubcoreMesh(core_axis_name="c", subcore_axis_name="s")
    return pl.kernel(
        _topk_sc_body,
        out_type=jax.ShapeDtypeStruct((32, 16), jnp.int32),
        mesh=mesh,
        scratch_types=[
            pltpu.VMEM((2, _TK), jnp.int32),
            pltpu.VMEM((256,), jnp.int32),
            pltpu.VMEM((16,), jnp.int32),
        ],
        compiler_params=pltpu.CompilerParams(needs_layout_passes=False),
    )(sbits)


_NP = _H // 2


def _attn_body(q_ref, k_ref, v_ref, s_ref, t_ref, wo_ref, b_ref, o_ref):
    j = pl.program_id(1)
    thr = t_ref[0][:, 0:1]
    mask = s_ref[0] >= thr
    q2 = q_ref[0]
    tile = jnp.concatenate([q2, q2], axis=0)
    row_id = lax.broadcasted_iota(jnp.int32, (2 * _TQ, 128), 0)
    lane_id = lax.broadcasted_iota(jnp.int32, (2 * _TQ, 128), 1)
    qp = jnp.where((row_id // _TQ) == (lane_id // _DH), tile, 0.0).astype(
        jnp.bfloat16
    )
    k2 = k_ref[0]
    att = lax.dot_general(
        qp, k2, (((1,), (1,)), ((), ())), preferred_element_type=jnp.float32
    ) * (1.0 / math.sqrt(_DH))
    mask2 = jnp.concatenate([mask, mask], axis=0)
    am = jnp.where(mask2, att, -1e30)
    m = jnp.max(am, axis=-1, keepdims=True)
    e = jnp.exp(am - m)
    ssum = jnp.sum(e, axis=-1, keepdims=True)
    aw = (e / ssum).astype(jnp.bfloat16)
    v2 = v_ref[0]
    pv = jnp.dot(aw, v2, preferred_element_type=jnp.float32)
    row_id = lax.broadcasted_iota(jnp.int32, (2 * _TQ, 128), 0)
    lane_id = lax.broadcasted_iota(jnp.int32, (2 * _TQ, 128), 1)
    keep = (row_id < _TQ) == (lane_id < _DH)
    pvc = jnp.where(keep, pv, 0.0)
    c2 = lax.dot_general(
        pvc, wo_ref[...], _DNT, preferred_element_type=jnp.float32
    )
    contrib = c2[0:_TQ, :] + c2[_TQ : 2 * _TQ, :]

    @pl.when(j == 0)
    def _():
        o_ref[0] = contrib + b_ref[...]

    @pl.when(j != 0)
    def _():
        o_ref[0] += contrib


def _attn(q_h, kv, scores, thr_b, wo, bo):
    grid = (_B, _NP)
    return pl.pallas_call(
        _attn_body,
        grid=grid,
        in_specs=[
            pl.BlockSpec((1, _TQ, 128), lambda b, j: (b, 0, j)),
            pl.BlockSpec((1, _TK, 128), lambda b, j: (b, 0, j)),
            pl.BlockSpec((1, _TK, 128), lambda b, j: (b, 0, j + _NP)),
            pl.BlockSpec((1, _TQ, _TK), lambda b, j: (b, 0, 0)),
            pl.BlockSpec((1, _TQ, 128), lambda b, j: (b, 0, 0)),
            pl.BlockSpec((_C, 128), lambda b, j: (0, j)),
            pl.BlockSpec((1, _C), lambda b, j: (0, 0)),
        ],
        out_specs=pl.BlockSpec((1, _TQ, _C), lambda b, j: (b, 0, 0)),
        out_shape=jax.ShapeDtypeStruct((_B, _TQ, _C), jnp.float32),
        compiler_params=pltpu.CompilerParams(
            dimension_semantics=("parallel", "arbitrary"),
        ),
    )(q_h, kv, kv, scores, thr_b, wo, bo)


def kernel(query, key, value, Wqi, Wki, head_weights, Wqkv, bqkv, Wout, bout):
    del value
    w_q = jax.nn.softmax(head_weights).astype(jnp.bfloat16).astype(jnp.float32)

    qf = query.reshape(_B * _TQ, _C)
    iq, Q = _proj_q(qf, Wqi, Wqkv[:_C], bqkv[:_C].reshape(1, _C))
    scores, sbits = _scores_k(key, Wki, iq.reshape(_B, _TQ, _HI * _DI), w_q)

    tbits = _topk_sc(sbits)
    thr = lax.bitcast_convert_type(tbits, jnp.float32)[:, :2].reshape(_B, _TQ, 1)
    thr_b = jnp.broadcast_to(thr, (_B, _TQ, 128))

    kv = _kv_k(key, Wqkv[_C:], bqkv[_C:].reshape(1, 2 * _C))

    out = _attn(
        Q.reshape(_B, _TQ, _C), kv, scores, thr_b, Wout, bout.reshape(1, _C)
    )
    return out
